# baseline (speedup 1.0000x reference)
import functools

import jax
import jax.numpy as jnp
from jax import lax
from jax.experimental import pallas as pl
from jax.experimental.pallas import tpu as pltpu
from jax.experimental.pallas import tpu_sc as plsc

D_MODEL = 64
NC = 2
NS = 16
NW = NC * NS

IDX_ROW = 128
IB = 4
CHUNK = IB * IDX_ROW


def _body(blocks, x_hbm, table_hbm, out_hbm, idx_v, rows_v, gsem):
    wid = lax.axis_index("s") * NC + lax.axis_index("c")
    row0 = wid * (blocks * IB)

    def blk(b, carry):
        r0 = row0 + b * IB
        pltpu.sync_copy(x_hbm.at[pl.ds(r0, IB)], idx_v)
        copies = [
            pltpu.async_copy(
                table_hbm.at[idx_v.at[j]],
                rows_v.at[pl.ds(j * IDX_ROW, IDX_ROW)],
                gsem,
            )
            for j in range(IB)
        ]
        for c in copies:
            c.wait()
        pltpu.sync_copy(rows_v, out_hbm.at[pl.ds(r0 * IDX_ROW, CHUNK)])
        return carry

    lax.fori_loop(0, blocks, blk, 0)


@jax.jit
def kernel(x, table):
    b, l = x.shape
    n = b * l
    assert n % (NW * CHUNK) == 0
    blocks = n // (NW * CHUNK)
    x2d = x.reshape(n // IDX_ROW, IDX_ROW).astype(jnp.int32)

    mesh = plsc.VectorSubcoreMesh(
        core_axis_name="c", subcore_axis_name="s", num_cores=NC, num_subcores=NS
    )
    out = pl.kernel(
        functools.partial(_body, blocks),
        out_type=jax.ShapeDtypeStruct((n, D_MODEL), jnp.float32),
        mesh=mesh,
        scratch_types=[
            pltpu.VMEM((IB, IDX_ROW), jnp.int32),
            pltpu.VMEM((CHUNK, D_MODEL), jnp.float32),
            pltpu.SemaphoreType.DMA,
        ],
        compiler_params=pltpu.CompilerParams(use_tc_tiling_on_sc=False),
    )(x2d, table)
    return out.reshape(b, l, D_MODEL)


# speedup vs baseline: 1.0413x; 1.0413x over previous
import functools

import jax
import jax.numpy as jnp
from jax import lax
from jax.experimental import pallas as pl
from jax.experimental.pallas import tpu as pltpu
from jax.experimental.pallas import tpu_sc as plsc

D_MODEL = 64
NC = 2
NS = 16
NW = NC * NS

IDX_ROW = 128
IB = 4
CHUNK = IB * IDX_ROW


def _body(blocks, x_hbm, table_hbm, out_hbm, idx_v, rows0, rows1, gsem0, gsem1,
          ssem0, ssem1):
    wid = lax.axis_index("s") * NC + lax.axis_index("c")
    row0 = wid * (blocks * IB)
    rows = (rows0, rows1)
    gsem = (gsem0, gsem1)
    ssem = (ssem0, ssem1)

    pltpu.sync_copy(x_hbm.at[pl.ds(row0, blocks * IB)], idx_v)

    def fire_gathers(g, p):
        return [
            pltpu.async_copy(
                table_hbm.at[idx_v.at[g * IB + j]],
                rows[p].at[pl.ds(j * IDX_ROW, IDX_ROW)],
                gsem[p],
            )
            for j in range(IB)
        ]

    def wait_gathers(g, p):
        for j in range(IB):
            pltpu.make_async_copy(
                table_hbm.at[idx_v.at[g * IB + j]],
                rows[p].at[pl.ds(j * IDX_ROW, IDX_ROW)],
                gsem[p],
            ).wait()

    def out_slice(g):
        return out_hbm.at[pl.ds((row0 + g * IB) * IDX_ROW, CHUNK)]

    def fire_store(g, p):
        pltpu.async_copy(rows[p], out_slice(g), ssem[p])

    def wait_store(g, p):
        pltpu.make_async_copy(rows[p], out_slice(g), ssem[p]).wait()

    fire_gathers(0, 0)
    fire_gathers(1, 1)
    wait_gathers(0, 0)
    fire_store(0, 0)

    @pl.loop(1, blocks - 1, step=2)
    def _(g0):
        for b in range(2):
            g = g0 + b
            p = (1 + b) % 2
            np_ = (b + 2) % 2
            wait_store(g - 1, np_)
            fire_gathers(g + 1, np_)
            wait_gathers(g, p)
            fire_store(g, p)

    last = blocks - 1
    wait_gathers(last, last % 2)
    fire_store(last, last % 2)
    wait_store(last - 1, (last - 1) % 2)
    wait_store(last, last % 2)


@jax.jit
def kernel(x, table):
    b, l = x.shape
    n = b * l
    assert n % (NW * CHUNK) == 0
    blocks = n // (NW * CHUNK)
    assert blocks % 2 == 0
    x2d = x.reshape(n // IDX_ROW, IDX_ROW).astype(jnp.int32)

    mesh = plsc.VectorSubcoreMesh(
        core_axis_name="c", subcore_axis_name="s", num_cores=NC, num_subcores=NS
    )
    out = pl.kernel(
        functools.partial(_body, blocks),
        out_type=jax.ShapeDtypeStruct((n, D_MODEL), jnp.float32),
        mesh=mesh,
        scratch_types=[
            pltpu.VMEM((blocks * IB, IDX_ROW), jnp.int32),
            pltpu.VMEM((CHUNK, D_MODEL), jnp.float32),
            pltpu.VMEM((CHUNK, D_MODEL), jnp.float32),
            pltpu.SemaphoreType.DMA,
            pltpu.SemaphoreType.DMA,
            pltpu.SemaphoreType.DMA,
            pltpu.SemaphoreType.DMA,
        ],
        compiler_params=pltpu.CompilerParams(use_tc_tiling_on_sc=False),
    )(x2d, table)
    return out.reshape(b, l, D_MODEL)


# speedup vs baseline: 1.0451x; 1.0036x over previous
import functools

import jax
import jax.numpy as jnp
from jax import lax
from jax.experimental import pallas as pl
from jax.experimental.pallas import tpu as pltpu
from jax.experimental.pallas import tpu_sc as plsc

D_MODEL = 64
NC = 2
NS = 16
NW = NC * NS

NB = 2
SPLITS = (0, 128)


def _body(l, bpw, blocks, x_hbm, table_hbm, out_hbm, idx_v, rows0, rows1,
          gsem0, gsem1, ssem0, ssem1):
    wid = lax.axis_index("s") * NC + lax.axis_index("c")
    b0 = wid * bpw
    rows = (rows0, rows1)
    gsem = (gsem0, gsem1)
    ssem = (ssem0, ssem1)
    widths = [SPLITS[i + 1] - SPLITS[i] if i + 1 < len(SPLITS) else l - SPLITS[i]
              for i in range(len(SPLITS))]

    pltpu.sync_copy(x_hbm.at[pl.ds(b0, bpw)], idx_v)

    def gather_copies(g, p):
        return [
            pltpu.make_async_copy(
                table_hbm.at[idx_v.at[g * NB + i, pl.ds(s, w)]],
                rows[p].at[i, pl.ds(s, w)],
                gsem[p],
            )
            for i in range(NB)
            for s, w in zip(SPLITS, widths)
        ]

    def fire_gathers(g, p):
        for c in gather_copies(g, p):
            c.start()

    def wait_gathers(g, p):
        for c in gather_copies(g, p):
            c.wait()

    def store_copy(g, p):
        return pltpu.make_async_copy(
            rows[p], out_hbm.at[pl.ds(b0 + g * NB, NB)], ssem[p]
        )

    fire_gathers(0, 0)
    fire_gathers(1, 1)
    wait_gathers(0, 0)
    store_copy(0, 0).start()

    @pl.loop(1, blocks - 1, step=2)
    def _(g0):
        for b in range(2):
            g = g0 + b
            p = (1 + b) % 2
            np_ = b % 2
            store_copy(g - 1, np_).wait()
            fire_gathers(g + 1, np_)
            wait_gathers(g, p)
            store_copy(g, p).start()

    last = blocks - 1
    wait_gathers(last, last % 2)
    store_copy(last, last % 2).start()
    store_copy(last - 1, (last - 1) % 2).wait()
    store_copy(last, last % 2).wait()


@jax.jit
def kernel(x, table):
    b, l = x.shape
    assert b % (NW * NB) == 0
    bpw = b // NW
    blocks = bpw // NB
    assert blocks % 2 == 0
    xi = x.astype(jnp.int32)

    mesh = plsc.VectorSubcoreMesh(
        core_axis_name="c", subcore_axis_name="s", num_cores=NC, num_subcores=NS
    )
    return pl.kernel(
        functools.partial(_body, l, bpw, blocks),
        out_type=jax.ShapeDtypeStruct((b, l, D_MODEL), jnp.float32),
        mesh=mesh,
        scratch_types=[
            pltpu.VMEM((b // NW, l), jnp.int32),
            pltpu.VMEM((NB, l, D_MODEL), jnp.float32),
            pltpu.VMEM((NB, l, D_MODEL), jnp.float32),
            pltpu.SemaphoreType.DMA,
            pltpu.SemaphoreType.DMA,
            pltpu.SemaphoreType.DMA,
            pltpu.SemaphoreType.DMA,
        ],
        compiler_params=pltpu.CompilerParams(use_tc_tiling_on_sc=False),
    )(xi, table)


# speedup vs baseline: 1.0968x; 1.0495x over previous
import functools

import jax
import jax.numpy as jnp
from jax import lax
from jax.experimental import pallas as pl
from jax.experimental.pallas import tpu as pltpu
from jax.experimental.pallas import tpu_sc as plsc

D_MODEL = 64
NC = 2
NS = 16
NW = NC * NS

NB = 2
SPLITS = (0, 128)


def _body(l, bpw, blocks, x_hbm, table_hbm, out_hbm, idx_v, rows0, rows1,
          gsem0, gsem1, ssem0, ssem1):
    wid = lax.axis_index("s") * NC + lax.axis_index("c")
    b0 = wid * bpw
    rows = (rows0, rows1)
    gsem = (gsem0, gsem1)
    ssem = (ssem0, ssem1)
    widths = [SPLITS[i + 1] - SPLITS[i] if i + 1 < len(SPLITS) else l - SPLITS[i]
              for i in range(len(SPLITS))]

    pltpu.sync_copy(x_hbm.at[pl.ds(b0, bpw)], idx_v)

    def gather_copies(g, p):
        return [
            pltpu.make_async_copy(
                table_hbm.at[idx_v.at[g * NB + i, pl.ds(s, w)]],
                rows[p].at[i, pl.ds(s, w)],
                gsem[p],
            )
            for i in range(NB)
            for s, w in zip(SPLITS, widths)
        ]

    def fire_gathers(g, p):
        for c in gather_copies(g, p):
            c.start()

    def wait_gathers(g, p):
        for c in gather_copies(g, p):
            c.wait()

    def store_copy(g, p):
        return pltpu.make_async_copy(
            rows[p], out_hbm.at[pl.ds(b0 + g * NB, NB)], ssem[p]
        )

    fire_gathers(0, 0)
    fire_gathers(1, 1)
    wait_gathers(0, 0)
    store_copy(0, 0).start()

    @pl.loop(1, blocks - 1, step=2)
    def _(g0):
        for b in range(2):
            g = g0 + b
            p = (1 + b) % 2
            np_ = b % 2
            store_copy(g - 1, np_).wait()
            fire_gathers(g + 1, np_)
            wait_gathers(g, p)
            store_copy(g, p).start()

    last = blocks - 1
    wait_gathers(last, last % 2)
    store_copy(last, last % 2).start()
    store_copy(last - 1, (last - 1) % 2).wait()
    store_copy(last, last % 2).wait()


@jax.jit
def kernel(x, table):
    b, l = x.shape
    assert b % (NW * NB) == 0
    bpw = b // NW
    blocks = bpw // NB
    assert blocks % 2 == 0
    xi = x.astype(jnp.int32) * 2
    v = table.shape[0]
    tpad = jnp.pad(table, ((0, 0), (0, 128 - D_MODEL)))
    tv = tpad.reshape(2 * v, D_MODEL)

    mesh = plsc.VectorSubcoreMesh(
        core_axis_name="c", subcore_axis_name="s", num_cores=NC, num_subcores=NS
    )
    return pl.kernel(
        functools.partial(_body, l, bpw, blocks),
        out_type=jax.ShapeDtypeStruct((b, l, D_MODEL), jnp.float32),
        mesh=mesh,
        scratch_types=[
            pltpu.VMEM((b // NW, l), jnp.int32),
            pltpu.VMEM((NB, l, D_MODEL), jnp.float32),
            pltpu.VMEM((NB, l, D_MODEL), jnp.float32),
            pltpu.SemaphoreType.DMA,
            pltpu.SemaphoreType.DMA,
            pltpu.SemaphoreType.DMA,
            pltpu.SemaphoreType.DMA,
        ],
        compiler_params=pltpu.CompilerParams(use_tc_tiling_on_sc=False),
    )(xi, tv)


# speedup vs baseline: 1.4861x; 1.3549x over previous
import functools

import jax
import jax.numpy as jnp
from jax import lax
from jax.experimental import pallas as pl
from jax.experimental.pallas import tpu as pltpu
from jax.experimental.pallas import tpu_sc as plsc

D_MODEL = 64
NC = 2
NS = 16
NW = NC * NS

NB = 2
SPLITS = (0, 128)


def _body(l, bpw, blocks, x_hbm, table_hbm, out_hbm, idx_v, rows0, rows1,
          gsem0, gsem1, ssem0, ssem1):
    wid = lax.axis_index("s") * NC + lax.axis_index("c")
    b0 = wid * bpw
    rows = (rows0, rows1)
    gsem = (gsem0, gsem1)
    ssem = (ssem0, ssem1)
    widths = [SPLITS[i + 1] - SPLITS[i] if i + 1 < len(SPLITS) else l - SPLITS[i]
              for i in range(len(SPLITS))]

    pltpu.sync_copy(x_hbm.at[pl.ds(b0, bpw)], idx_v)

    def gather_copies(g, p):
        return [
            pltpu.make_async_copy(
                table_hbm.at[idx_v.at[g * NB + i, pl.ds(s, w)]],
                rows[p].at[pl.ds(i * l + s, w)],
                gsem[p],
            )
            for i in range(NB)
            for s, w in zip(SPLITS, widths)
        ]

    def fire_gathers(g, p):
        for c in gather_copies(g, p):
            c.start()

    def wait_gathers(g, p):
        for c in gather_copies(g, p):
            c.wait()

    def store_copy(g, p):
        return pltpu.make_async_copy(
            rows[p],
            out_hbm.at[pl.ds((b0 + g * NB) * l, NB * l), pl.ds(0, D_MODEL)],
            ssem[p],
        )

    fire_gathers(0, 0)
    fire_gathers(1, 1)
    wait_gathers(0, 0)
    store_copy(0, 0).start()

    @pl.loop(1, blocks - 1, step=2)
    def _(g0):
        for b in range(2):
            g = g0 + b
            p = (1 + b) % 2
            np_ = b % 2
            store_copy(g - 1, np_).wait()
            fire_gathers(g + 1, np_)
            wait_gathers(g, p)
            store_copy(g, p).start()

    last = blocks - 1
    wait_gathers(last, last % 2)
    store_copy(last, last % 2).start()
    store_copy(last - 1, (last - 1) % 2).wait()
    store_copy(last, last % 2).wait()


@jax.jit
def kernel(x, table):
    b, l = x.shape
    assert b % (NW * NB) == 0
    bpw = b // NW
    blocks = bpw // NB
    assert blocks % 2 == 0
    xi = x.astype(jnp.int32) * 2
    v = table.shape[0]
    tpad = jnp.pad(table, ((0, 0), (0, 128 - D_MODEL)))
    tv = tpad.reshape(2 * v, D_MODEL)

    mesh = plsc.VectorSubcoreMesh(
        core_axis_name="c", subcore_axis_name="s", num_cores=NC, num_subcores=NS
    )
    out = pl.kernel(
        functools.partial(_body, l, bpw, blocks),
        out_type=jax.ShapeDtypeStruct((b * l, 128), jnp.float32),
        mesh=mesh,
        scratch_types=[
            pltpu.VMEM((b // NW, l), jnp.int32),
            pltpu.VMEM((NB * l, D_MODEL), jnp.float32),
            pltpu.VMEM((NB * l, D_MODEL), jnp.float32),
            pltpu.SemaphoreType.DMA,
            pltpu.SemaphoreType.DMA,
            pltpu.SemaphoreType.DMA,
            pltpu.SemaphoreType.DMA,
        ],
        compiler_params=pltpu.CompilerParams(use_tc_tiling_on_sc=False),
    )(xi, tv)
    return out[:, :D_MODEL].reshape(b, l, D_MODEL)
